# Initial kernel scaffold; baseline (speedup 1.0000x reference)
import jax
import jax.numpy as jnp
from jax.experimental import pallas as pl


def kernel(node_embeddings, user_weights, item_weights, user_indices, item_indices, user_walks, item_walks):
    raise NotImplementedError("write your pallas kernel here")


# speedup vs baseline: 3.3474x; 3.3474x over previous
import functools

import jax
import jax.numpy as jnp
from jax import lax
from jax.experimental import pallas as pl
from jax.experimental.pallas import tpu as pltpu
from jax.experimental.pallas import tpu_sc as plsc

D = 200
S = 25
K = 3
R = 1 + S * K
B = 4096
NC, NS, L = 2, 16, 16
NW = NC * NS
PER_W = B // NW
COLS = tuple(c * L for c in range(12)) + (D - L,)


def _sc_scores(emb, wpad, idx_u, idx_i):
    mesh = plsc.VectorSubcoreMesh(core_axis_name="c", subcore_axis_name="s")

    @functools.partial(
        pl.kernel,
        out_type=jax.ShapeDtypeStruct((NW, PER_W), jnp.float32),
        mesh=mesh,
        compiler_params=pltpu.CompilerParams(use_tc_tiling_on_sc=False,
                                             needs_layout_passes=False),
        scratch_types=[
            pltpu.VMEM((L,), jnp.float32),
            pltpu.VMEM((PER_W, R), jnp.int32),
            pltpu.VMEM((PER_W, R), jnp.int32),
            pltpu.VMEM((2 * R, D), jnp.float32),
            pltpu.VMEM((PER_W,), jnp.float32),
            pltpu.SemaphoreType.DMA,
        ],
    )
    def body(emb_hbm, w_hbm, idxu_hbm, idxi_hbm, out_hbm,
             w_v, idxu_v, idxi_v, rows_v, out_v, sem):
        wid = lax.axis_index("s") * NC + lax.axis_index("c")
        pltpu.sync_copy(w_hbm, w_v)
        pltpu.sync_copy(idxu_hbm.at[wid], idxu_v)
        pltpu.sync_copy(idxi_hbm.at[wid], idxi_v)

        wv = w_v[...]
        e = jnp.exp(wv - jnp.max(wv))
        wn = e / jnp.full((L,), jnp.sum(e), jnp.float32)
        lanes = lax.iota(jnp.int32, L)
        zero = jnp.zeros((L,), jnp.float32)
        w0 = jnp.sum(jnp.where(lanes == 0, wn, zero))
        w1 = jnp.sum(jnp.where(lanes == 1, wn, zero)) * (1.0 / S)
        w2 = jnp.sum(jnp.where(lanes == 2, wn, zero)) * (1.0 / S)
        w3 = jnp.sum(jnp.where(lanes == 3, wn, zero)) * (1.0 / S)
        tail_mask = lanes >= (L - (D - COLS[11] - L))

        def side_repr(base):
            e0 = [rows_v[base, pl.ds(col, L)] for col in COLS]

            def group(first_row):
                def gbody(r, accs):
                    row = first_row + r
                    return tuple(acc + rows_v[row, pl.ds(col, L)]
                                 for acc, col in zip(accs, COLS))
                init = tuple(zero for _ in COLS)
                return lax.fori_loop(0, S, gbody, init)

            g1 = group(base + 1)
            g2 = group(base + 1 + S)
            g3 = group(base + 1 + 2 * S)
            return [w0 * a + w1 * b + w2 * c + w3 * d
                    for a, b, c, d in zip(e0, g1, g2, g3)]

        def elem(n, carry):
            cu = pltpu.async_copy(emb_hbm.at[idxu_v.at[n]],
                                  rows_v.at[pl.ds(0, R)], sem)
            ci = pltpu.async_copy(emb_hbm.at[idxi_v.at[n]],
                                  rows_v.at[pl.ds(R, R)], sem)
            cu.wait()
            ci.wait()
            u = side_repr(0)
            v = side_repr(R)
            p = zero
            for c in range(12):
                p = p + u[c] * v[c]
            tail = u[12] * v[12]
            p = p + jnp.where(tail_mask, tail, zero)
            dot = jnp.sum(p)
            plsc.store_scatter(out_v, [jnp.full((L,), n, jnp.int32)],
                               jnp.full((L,), dot, jnp.float32),
                               mask=lanes == 0)
            return carry

        lax.fori_loop(0, PER_W, elem, 0)
        pltpu.sync_copy(out_v, out_hbm.at[wid])

    return body(emb, wpad, idx_u, idx_i)


def kernel(node_embeddings, user_weights, item_weights,
           user_indices, item_indices, user_walks, item_walks):
    del item_weights
    wpad = jnp.pad(user_weights, (0, L - user_weights.shape[0]),
                   constant_values=-jnp.inf)

    def pack_idx(indices, walks):
        steps = walks[:, :, 1:].transpose(0, 2, 1).reshape(B, S * K)
        idx = jnp.concatenate([indices[:, None], steps], axis=1)
        return idx.astype(jnp.int32).reshape(NW, PER_W, R)

    idx_u = pack_idx(user_indices, user_walks)
    idx_i = pack_idx(item_indices, item_walks)
    out = _sc_scores(node_embeddings, wpad, idx_u, idx_i)
    return out.reshape(B)


# speedup vs baseline: 4.1635x; 1.2438x over previous
import functools

import jax
import jax.numpy as jnp
from jax import lax
from jax.experimental import pallas as pl
from jax.experimental.pallas import tpu as pltpu
from jax.experimental.pallas import tpu_sc as plsc

D = 200
S = 25
K = 3
R = 1 + S * K
B = 4096
NC, NS, L = 2, 16, 16
NW = NC * NS
PER_W = B // NW
COLS = tuple(c * L for c in range(12)) + (D - L,)


def _sc_scores(emb, wpad, idx_u, idx_i):
    mesh = plsc.VectorSubcoreMesh(core_axis_name="c", subcore_axis_name="s")

    @functools.partial(
        pl.kernel,
        out_type=jax.ShapeDtypeStruct((NW, PER_W), jnp.float32),
        mesh=mesh,
        compiler_params=pltpu.CompilerParams(use_tc_tiling_on_sc=False,
                                             needs_layout_passes=False),
        scratch_types=[
            pltpu.VMEM((L,), jnp.float32),
            pltpu.VMEM((PER_W, R), jnp.int32),
            pltpu.VMEM((PER_W, R), jnp.int32),
            pltpu.VMEM((2 * R, D), jnp.float32),
            pltpu.VMEM((2 * R, D), jnp.float32),
            pltpu.VMEM((PER_W,), jnp.float32),
            pltpu.SemaphoreType.DMA,
            pltpu.SemaphoreType.DMA,
        ],
    )
    def body(emb_hbm, w_hbm, idxu_hbm, idxi_hbm, out_hbm,
             w_v, idxu_v, idxi_v, rows_a, rows_b, out_v, sem_a, sem_b):
        wid = lax.axis_index("s") * NC + lax.axis_index("c")
        pltpu.sync_copy(w_hbm, w_v)
        pltpu.sync_copy(idxu_hbm.at[wid], idxu_v)
        pltpu.sync_copy(idxi_hbm.at[wid], idxi_v)

        wv = w_v[...]
        e = jnp.exp(wv - jnp.max(wv))
        wn = e / jnp.full((L,), jnp.sum(e), jnp.float32)
        lanes = lax.iota(jnp.int32, L)
        zero = jnp.zeros((L,), jnp.float32)
        w0 = jnp.sum(jnp.where(lanes == 0, wn, zero))
        w1 = jnp.sum(jnp.where(lanes == 1, wn, zero)) * (1.0 / S)
        w2 = jnp.sum(jnp.where(lanes == 2, wn, zero)) * (1.0 / S)
        w3 = jnp.sum(jnp.where(lanes == 3, wn, zero)) * (1.0 / S)
        tail_mask = lanes >= (L - (D - COLS[11] - L))

        def issue(n, rows_v, sem):
            pltpu.async_copy(emb_hbm.at[idxu_v.at[n]],
                             rows_v.at[pl.ds(0, R)], sem)
            pltpu.async_copy(emb_hbm.at[idxi_v.at[n]],
                             rows_v.at[pl.ds(R, R)], sem)

        def drain(rows_v, sem):
            pltpu.make_async_copy(emb_hbm.at[pl.ds(0, 2 * R)], rows_v,
                                  sem).wait()

        def side_repr(rows_v, base):
            e0 = [rows_v[base, pl.ds(col, L)] for col in COLS]

            def group(first_row):
                def gbody(r, accs):
                    row = first_row + r
                    return tuple(acc + rows_v[row, pl.ds(col, L)]
                                 for acc, col in zip(accs, COLS))
                init = tuple(zero for _ in COLS)
                return lax.fori_loop(0, S, gbody, init)

            g1 = group(base + 1)
            g2 = group(base + 1 + S)
            g3 = group(base + 1 + 2 * S)
            return [w0 * a + w1 * b + w2 * c + w3 * d
                    for a, b, c, d in zip(e0, g1, g2, g3)]

        def compute(n, rows_v):
            u = side_repr(rows_v, 0)
            v = side_repr(rows_v, R)
            p = zero
            for c in range(12):
                p = p + u[c] * v[c]
            tail = u[12] * v[12]
            p = p + jnp.where(tail_mask, tail, zero)
            dot = jnp.sum(p)
            plsc.store_scatter(out_v, [jnp.full((L,), n, jnp.int32)],
                               jnp.full((L,), dot, jnp.float32),
                               mask=lanes == 0)

        issue(0, rows_a, sem_a)
        issue(1, rows_b, sem_b)

        def grp(g, carry):
            for n, rows_v, sem in ((2 * g, rows_a, sem_a),
                                   (2 * g + 1, rows_b, sem_b)):
                drain(rows_v, sem)
                compute(n, rows_v)

                @pl.when(n + 2 < PER_W)
                def _():
                    issue(n + 2, rows_v, sem)
            return carry

        lax.fori_loop(0, PER_W // 2, grp, 0)
        pltpu.sync_copy(out_v, out_hbm.at[wid])

    return body(emb, wpad, idx_u, idx_i)


def kernel(node_embeddings, user_weights, item_weights,
           user_indices, item_indices, user_walks, item_walks):
    del item_weights
    wpad = jnp.pad(user_weights, (0, L - user_weights.shape[0]),
                   constant_values=-jnp.inf)

    def pack_idx(indices, walks):
        steps = walks[:, :, 1:].transpose(0, 2, 1).reshape(B, S * K)
        idx = jnp.concatenate([indices[:, None], steps], axis=1)
        return idx.astype(jnp.int32).reshape(NW, PER_W, R)

    idx_u = pack_idx(user_indices, user_walks)
    idx_i = pack_idx(item_indices, item_walks)
    out = _sc_scores(node_embeddings, wpad, idx_u, idx_i)
    return out.reshape(B)
